# baseline (speedup 1.0000x reference)
import jax
import jax.numpy as jnp
from jax.experimental import pallas as pl
from jax.experimental.pallas import tpu as pltpu

T = 2048
D = 1024
E = 8
K = 2
I = 512
ISH = 1024
RSF = 2.5

NEG = -1e30
BF = jnp.bfloat16
F32 = jnp.float32

NT = (((1,), (1,)), ((), ()))


def _router_body(x_ref, gwt_ref, bias_ref, comb_ref, idx_ref):
    logits = jnp.dot(x_ref[...], gwt_ref[...],
                     preferred_element_type=F32)[:, :E]
    scores = jax.nn.sigmoid(logits)
    sc = scores + bias_ref[...]
    B = scores.shape[0]

    def top2sum(g):
        s = None
        for i in range(4):
            for j in range(i + 1, 4):
                p = g[:, i:i + 1] + g[:, j:j + 1]
                s = p if s is None else jnp.maximum(s, p)
        return s

    gs0 = top2sum(sc[:, 0:4])
    gs1 = top2sum(sc[:, 4:8])
    chosen = jnp.where(gs0 >= gs1, 0, 1)
    lane = jax.lax.broadcasted_iota(jnp.int32, (B, E), 1)
    emask = (lane // 4) == chosen
    masked = jnp.where(emask, sc, NEG)
    m1 = jnp.max(masked, axis=1, keepdims=True)
    i1 = jnp.min(jnp.where(masked == m1, lane, E), axis=1, keepdims=True)
    masked2 = jnp.where(lane == i1, NEG, masked)
    m2 = jnp.max(masked2, axis=1, keepdims=True)
    i2 = jnp.min(jnp.where(masked2 == m2, lane, E), axis=1, keepdims=True)
    selmask = jnp.logical_or(lane == i1, lane == i2)
    wsel = jnp.where(selmask, scores, 0.0)
    wsum = jnp.sum(wsel, axis=1, keepdims=True) + 1e-20
    comb = wsel * (RSF / wsum)
    lane16 = jax.lax.broadcasted_iota(jnp.int32, (B, 16), 1)
    shared_cols = jnp.logical_and(lane16 >= E, lane16 < E + 2)
    comb_ref[...] = jnp.where(
        shared_cols, 1.0,
        jnp.where(lane16 < E, jnp.pad(comb, ((0, 0), (0, 8))), 0.0))
    idx_ref[...] = jnp.concatenate([i1, i2], axis=1)


def _router(x, gate_w, bias):
    gwt = jnp.zeros((D, 128), F32).at[:, :E].set(gate_w.T)
    bias2 = bias.reshape(1, E)
    BT = 512
    return pl.pallas_call(
        _router_body,
        grid=(T // BT,),
        in_specs=[
            pl.BlockSpec((BT, D), lambda b: (b, 0)),
            pl.BlockSpec((D, 128), lambda b: (0, 0)),
            pl.BlockSpec((1, E), lambda b: (0, 0)),
        ],
        out_specs=[
            pl.BlockSpec((BT, 16), lambda b: (b, 0)),
            pl.BlockSpec((BT, K), lambda b: (b, 0)),
        ],
        out_shape=[
            jax.ShapeDtypeStruct((T, 16), F32),
            jax.ShapeDtypeStruct((T, K), jnp.int32),
        ],
    )(x, gwt, bias2)


def _shared_body(x_ref, wgu_ref, wd_ref, out_ref):
    gu = jax.lax.dot_general(x_ref[...], wgu_ref[...],
                             NT, preferred_element_type=F32)
    h = jax.nn.silu(gu[:, :ISH]) * gu[:, ISH:]
    out_ref[...] = jax.lax.dot_general(h.astype(BF), wd_ref[...],
                                       NT, preferred_element_type=F32)


def _shared(x, w_gu, w_d):
    BT = 512
    return pl.pallas_call(
        _shared_body,
        grid=(T // BT,),
        in_specs=[
            pl.BlockSpec((BT, D), lambda b: (b, 0)),
            pl.BlockSpec((2 * ISH, D), lambda b: (0, 0)),
            pl.BlockSpec((D, ISH), lambda b: (0, 0)),
        ],
        out_specs=pl.BlockSpec((BT, D), lambda b: (b, 0)),
        out_shape=jax.ShapeDtypeStruct((T, D), F32),
    )(x, w_gu, w_d)


def _routed_body(x_ref, wgu_ref, wd_ref, comb_ref, shared_ref, out_ref):
    e = pl.program_id(0)
    b = pl.program_id(1)
    BT = 512
    rows = pl.ds(b * BT, BT)
    xb = x_ref[rows, :]
    lane = jax.lax.broadcasted_iota(jnp.int32, (BT, 16), 1)
    col = jnp.sum(jnp.where(lane == e, comb_ref[rows, :], 0.0), axis=1,
                  keepdims=True)
    gu = jax.lax.dot_general(xb, wgu_ref[0],
                             NT, preferred_element_type=F32)
    h = jax.nn.silu(gu[:, :I]) * gu[:, I:] * col
    y = jax.lax.dot_general(h.astype(BF), wd_ref[0],
                            NT, preferred_element_type=F32)

    @pl.when(e == 0)
    def _():
        out_ref[rows, :] = shared_ref[rows, :] + y

    @pl.when(e > 0)
    def _():
        out_ref[rows, :] = out_ref[rows, :] + y


def _routed(x, w_gu, w_d, comb, shared_out):
    NB = 4
    return pl.pallas_call(
        _routed_body,
        grid=(E, NB),
        in_specs=[
            pl.BlockSpec((T, D), lambda e, b: (0, 0)),
            pl.BlockSpec((1, 2 * I, D), lambda e, b: (e, 0, 0)),
            pl.BlockSpec((1, D, I), lambda e, b: (e, 0, 0)),
            pl.BlockSpec((T, 16), lambda e, b: (0, 0)),
            pl.BlockSpec((T, D), lambda e, b: (0, 0)),
        ],
        out_specs=pl.BlockSpec((T, D), lambda e, b: (0, 0)),
        out_shape=jax.ShapeDtypeStruct((T, D), F32),
    )(x, w_gu, w_d, comb, shared_out)


def kernel(x, max_num_tokens_per_gpu, gate_w, e_score_correction_bias,
           w_shared_gate_up, w_shared_down, w_expert_gate_up, w_expert_down):
    comb, _ = _router(x, gate_w, e_score_correction_bias)
    xb = x.astype(BF)
    shared_out = _shared(xb, w_shared_gate_up.astype(BF),
                         w_shared_down.astype(BF))
    return _routed(xb, w_expert_gate_up.astype(BF),
                   w_expert_down.astype(BF), comb, shared_out)


# speedup vs baseline: 1.2926x; 1.2926x over previous
import jax
import jax.numpy as jnp
from jax.experimental import pallas as pl
from jax.experimental.pallas import tpu as pltpu

T = 2048
D = 1024
E = 8
K = 2
I = 512
ISH = 1024
RSF = 2.5

NEG = -1e30
BF = jnp.bfloat16
F32 = jnp.float32

NT = (((1,), (1,)), ((), ()))


def _router_body(x_ref, gwt_ref, bias_ref, comb_ref, idx_ref):
    logits = jnp.dot(x_ref[...], gwt_ref[...],
                     preferred_element_type=F32)[:, :E]
    scores = jax.nn.sigmoid(logits)
    sc = scores + bias_ref[...]
    B = scores.shape[0]

    def top2sum(g):
        s = None
        for i in range(4):
            for j in range(i + 1, 4):
                p = g[:, i:i + 1] + g[:, j:j + 1]
                s = p if s is None else jnp.maximum(s, p)
        return s

    gs0 = top2sum(sc[:, 0:4])
    gs1 = top2sum(sc[:, 4:8])
    chosen = jnp.where(gs0 >= gs1, 0, 1)
    lane = jax.lax.broadcasted_iota(jnp.int32, (B, E), 1)
    emask = (lane // 4) == chosen
    masked = jnp.where(emask, sc, NEG)
    m1 = jnp.max(masked, axis=1, keepdims=True)
    i1 = jnp.min(jnp.where(masked == m1, lane, E), axis=1, keepdims=True)
    masked2 = jnp.where(lane == i1, NEG, masked)
    m2 = jnp.max(masked2, axis=1, keepdims=True)
    i2 = jnp.min(jnp.where(masked2 == m2, lane, E), axis=1, keepdims=True)
    selmask = jnp.logical_or(lane == i1, lane == i2)
    wsel = jnp.where(selmask, scores, 0.0)
    wsum = jnp.sum(wsel, axis=1, keepdims=True) + 1e-20
    comb = wsel * (RSF / wsum)
    lane16 = jax.lax.broadcasted_iota(jnp.int32, (B, 16), 1)
    shared_cols = jnp.logical_and(lane16 >= E, lane16 < E + 2)
    comb_ref[...] = jnp.where(
        shared_cols, 1.0,
        jnp.where(lane16 < E, jnp.pad(comb, ((0, 0), (0, 8))), 0.0))
    idx_ref[...] = jnp.concatenate([i1, i2], axis=1)


def _router(x, gate_w, bias):
    gwt = jnp.zeros((D, 128), F32).at[:, :E].set(gate_w.T)
    bias2 = bias.reshape(1, E)
    BT = 512
    return pl.pallas_call(
        _router_body,
        grid=(T // BT,),
        in_specs=[
            pl.BlockSpec((BT, D), lambda b: (b, 0)),
            pl.BlockSpec((D, 128), lambda b: (0, 0)),
            pl.BlockSpec((1, E), lambda b: (0, 0)),
        ],
        out_specs=[
            pl.BlockSpec((BT, 16), lambda b: (b, 0)),
            pl.BlockSpec((BT, K), lambda b: (b, 0)),
        ],
        out_shape=[
            jax.ShapeDtypeStruct((T, 16), F32),
            jax.ShapeDtypeStruct((T, K), jnp.int32),
        ],
    )(x, gwt, bias2)


def _shared_body(x_ref, wgu_ref, wd_ref, out_ref):
    gu = jax.lax.dot_general(x_ref[...], wgu_ref[...].astype(BF),
                             NT, preferred_element_type=F32)
    h = jax.nn.silu(gu[:, :ISH]) * gu[:, ISH:]
    out_ref[...] = jax.lax.dot_general(h.astype(BF), wd_ref[...].astype(BF),
                                       NT, preferred_element_type=F32)


def _shared(x, w_gu, w_d):
    BT = 512
    return pl.pallas_call(
        _shared_body,
        grid=(T // BT,),
        in_specs=[
            pl.BlockSpec((BT, D), lambda b: (b, 0)),
            pl.BlockSpec((2 * ISH, D), lambda b: (0, 0)),
            pl.BlockSpec((D, ISH), lambda b: (0, 0)),
        ],
        out_specs=pl.BlockSpec((BT, D), lambda b: (b, 0)),
        out_shape=jax.ShapeDtypeStruct((T, D), F32),
    )(x, w_gu, w_d)


def _routed_body(x_ref, wgu_ref, wd_ref, comb_ref, shared_ref, out_ref):
    e = pl.program_id(0)
    lane = jax.lax.broadcasted_iota(jnp.int32, (T, 16), 1)
    col = jnp.sum(jnp.where(lane == e, comb_ref[...], 0.0), axis=1,
                  keepdims=True)
    gu = jax.lax.dot_general(x_ref[...], wgu_ref[0].astype(BF),
                             NT, preferred_element_type=F32)
    h = jax.nn.silu(gu[:, :I]) * gu[:, I:] * col
    y = jax.lax.dot_general(h.astype(BF), wd_ref[0].astype(BF),
                            NT, preferred_element_type=F32)

    @pl.when(e == 0)
    def _():
        out_ref[...] = shared_ref[...] + y

    @pl.when(e > 0)
    def _():
        out_ref[...] = out_ref[...] + y


def _routed(x, w_gu, w_d, comb, shared_out):
    return pl.pallas_call(
        _routed_body,
        grid=(E,),
        in_specs=[
            pl.BlockSpec((T, D), lambda e: (0, 0)),
            pl.BlockSpec((1, 2 * I, D), lambda e: (e, 0, 0)),
            pl.BlockSpec((1, D, I), lambda e: (e, 0, 0)),
            pl.BlockSpec((T, 16), lambda e: (0, 0)),
            pl.BlockSpec((T, D), lambda e: (0, 0)),
        ],
        out_specs=pl.BlockSpec((T, D), lambda e: (0, 0)),
        out_shape=jax.ShapeDtypeStruct((T, D), F32),
    )(x, w_gu, w_d, comb, shared_out)


def kernel(x, max_num_tokens_per_gpu, gate_w, e_score_correction_bias,
           w_shared_gate_up, w_shared_down, w_expert_gate_up, w_expert_down):
    comb, _ = _router(x, gate_w, e_score_correction_bias)
    xb = x.astype(BF)
    shared_out = _shared(xb, w_shared_gate_up, w_shared_down)
    return _routed(xb, w_expert_gate_up, w_expert_down, comb, shared_out)


# speedup vs baseline: 1.3251x; 1.0252x over previous
import jax
import jax.numpy as jnp
from jax.experimental import pallas as pl
from jax.experimental.pallas import tpu as pltpu

T = 2048
D = 1024
E = 8
K = 2
I = 512
ISH = 1024
RSF = 2.5

NEG = -1e30
BF = jnp.bfloat16
F32 = jnp.float32

NT = (((1,), (1,)), ((), ()))


def _router_body(x_ref, gwt_ref, bias_ref, comb_ref, idx_ref):
    logits = jnp.dot(x_ref[...], gwt_ref[...],
                     preferred_element_type=F32)[:, :E]
    scores = jax.nn.sigmoid(logits)
    sc = scores + bias_ref[...]
    B = scores.shape[0]

    def top2sum(g):
        s = None
        for i in range(4):
            for j in range(i + 1, 4):
                p = g[:, i:i + 1] + g[:, j:j + 1]
                s = p if s is None else jnp.maximum(s, p)
        return s

    gs0 = top2sum(sc[:, 0:4])
    gs1 = top2sum(sc[:, 4:8])
    chosen = jnp.where(gs0 >= gs1, 0, 1)
    lane = jax.lax.broadcasted_iota(jnp.int32, (B, E), 1)
    emask = (lane // 4) == chosen
    masked = jnp.where(emask, sc, NEG)
    m1 = jnp.max(masked, axis=1, keepdims=True)
    i1 = jnp.min(jnp.where(masked == m1, lane, E), axis=1, keepdims=True)
    masked2 = jnp.where(lane == i1, NEG, masked)
    m2 = jnp.max(masked2, axis=1, keepdims=True)
    i2 = jnp.min(jnp.where(masked2 == m2, lane, E), axis=1, keepdims=True)
    selmask = jnp.logical_or(lane == i1, lane == i2)
    wsel = jnp.where(selmask, scores, 0.0)
    wsum = jnp.sum(wsel, axis=1, keepdims=True) + 1e-20
    comb = wsel * (RSF / wsum)
    lane16 = jax.lax.broadcasted_iota(jnp.int32, (B, 16), 1)
    shared_cols = jnp.logical_and(lane16 >= E, lane16 < E + 2)
    comb_ref[...] = jnp.where(
        shared_cols, 1.0,
        jnp.where(lane16 < E, jnp.pad(comb, ((0, 0), (0, 8))), 0.0))
    idx_ref[...] = jnp.concatenate([i1, i2], axis=1)


def _router(x, gate_w, bias):
    gwt = jnp.zeros((D, 128), F32).at[:, :E].set(gate_w.T)
    bias2 = bias.reshape(1, E)
    BT = 512
    return pl.pallas_call(
        _router_body,
        grid=(T // BT,),
        in_specs=[
            pl.BlockSpec((BT, D), lambda b: (b, 0)),
            pl.BlockSpec((D, 128), lambda b: (0, 0)),
            pl.BlockSpec((1, E), lambda b: (0, 0)),
        ],
        out_specs=[
            pl.BlockSpec((BT, 16), lambda b: (b, 0)),
            pl.BlockSpec((BT, K), lambda b: (b, 0)),
        ],
        out_shape=[
            jax.ShapeDtypeStruct((T, 16), F32),
            jax.ShapeDtypeStruct((T, K), jnp.int32),
        ],
    )(x, gwt, bias2)


def _shared_body(x_ref, wgu_ref, wd_ref, out_ref):
    gu = jax.lax.dot_general(x_ref[...], wgu_ref[...].astype(BF),
                             NT, preferred_element_type=F32)
    h = jax.nn.silu(gu[:, :ISH]) * gu[:, ISH:]
    out_ref[...] = jax.lax.dot_general(h.astype(BF), wd_ref[...].astype(BF),
                                       NT, preferred_element_type=F32)


def _shared(x, w_gu, w_d):
    BT = 512
    return pl.pallas_call(
        _shared_body,
        grid=(T // BT,),
        in_specs=[
            pl.BlockSpec((BT, D), lambda b: (b, 0)),
            pl.BlockSpec((2 * ISH, D), lambda b: (0, 0)),
            pl.BlockSpec((D, ISH), lambda b: (0, 0)),
        ],
        out_specs=pl.BlockSpec((BT, D), lambda b: (b, 0)),
        out_shape=jax.ShapeDtypeStruct((T, D), F32),
    )(x, w_gu, w_d)


def _moe_body(x_ref, sg_ref, su_ref, sd_ref, eg_ref, eu_ref, ed_ref,
              comb_ref, out_ref):
    e = pl.program_id(0)
    c = jnp.where(e < 2, e + 8, e - 2)
    lane = jax.lax.broadcasted_iota(jnp.int32, (T, 16), 1)
    col = jnp.sum(jnp.where(lane == c, comb_ref[...], 0.0), axis=1,
                  keepdims=True)

    def mlp(g_w, u_w, d_w):
        xb = x_ref[...]
        g = jax.lax.dot_general(xb, g_w.astype(BF), NT,
                                preferred_element_type=F32)
        u = jax.lax.dot_general(xb, u_w.astype(BF), NT,
                                preferred_element_type=F32)
        h = jax.nn.silu(g) * u * col
        y = jax.lax.dot_general(h.astype(BF), d_w.astype(BF), NT,
                                preferred_element_type=F32)

        @pl.when(e == 0)
        def _():
            out_ref[...] = y

        @pl.when(e > 0)
        def _():
            out_ref[...] = out_ref[...] + y

    @pl.when(e < 2)
    def _():
        mlp(sg_ref[...], su_ref[...], sd_ref[...])

    @pl.when(e >= 2)
    def _():
        mlp(eg_ref[0], eu_ref[0], ed_ref[0])


def _moe(x, w_sgu, w_sd, w_egu, w_ed, comb):
    sh = lambda e: (jnp.minimum(e, 1), 0)
    su = lambda e: (2 + jnp.minimum(e, 1), 0)
    sd = lambda e: (0, jnp.minimum(e, 1))
    ex = lambda e: jnp.maximum(e - 2, 0)
    return pl.pallas_call(
        _moe_body,
        grid=(E + 2,),
        in_specs=[
            pl.BlockSpec((T, D), lambda e: (0, 0)),
            pl.BlockSpec((I, D), sh),
            pl.BlockSpec((I, D), su),
            pl.BlockSpec((D, I), sd),
            pl.BlockSpec((1, I, D), lambda e: (ex(e), 0, 0)),
            pl.BlockSpec((1, I, D), lambda e: (ex(e), 1, 0)),
            pl.BlockSpec((1, D, I), lambda e: (ex(e), 0, 0)),
            pl.BlockSpec((T, 16), lambda e: (0, 0)),
        ],
        out_specs=pl.BlockSpec((T, D), lambda e: (0, 0)),
        out_shape=jax.ShapeDtypeStruct((T, D), F32),
    )(x, w_sgu, w_sgu, w_sd, w_egu, w_egu, w_ed, comb)


def kernel(x, max_num_tokens_per_gpu, gate_w, e_score_correction_bias,
           w_shared_gate_up, w_shared_down, w_expert_gate_up, w_expert_down):
    comb, _ = _router(x, gate_w, e_score_correction_bias)
    xb = x.astype(BF)
    return _moe(xb, w_shared_gate_up, w_shared_down,
                w_expert_gate_up, w_expert_down, comb)
